# baseline (speedup 1.0000x reference)
import functools

import jax
import jax.numpy as jnp
from jax import lax
from jax.experimental import pallas as pl
from jax.experimental.pallas import tpu as pltpu
from jax.experimental.pallas import tpu_sc as plsc

N = 10000
NP = 10240
F = 16
NC = 2
NS = 16
NW = NC * NS
JC = 1024
SLICE = NP // NS
NPK = NP // 8
NPK8 = NP // 16

_mesh = plsc.VectorSubcoreMesh(
    core_axis_name="c", subcore_axis_name="s", num_cores=NC, num_subcores=NS)
_sc_params = pltpu.CompilerParams(use_tc_tiling_on_sc=False)


def _make_sc_degree(sizes):
    offs = [sum(sizes[:g]) for g in range(len(sizes))]

    @functools.partial(
        pl.kernel,
        out_type=jax.ShapeDtypeStruct((NC, NP, F), jnp.float32),
        mesh=_mesh,
        compiler_params=_sc_params,
        scratch_types=[
            [pltpu.VMEM((sz,), jnp.int32) for sz in sizes],
            pltpu.VMEM((JC, F), jnp.float32),
            pltpu.VMEM_SHARED((NP, F), jnp.float32),
            pltpu.SemaphoreType.DMA,
        ],
    )
    def deg_kernel(edge_hbm, ones_hbm, zeros_hbm, out_hbm,
                   dstv, onesv, acc, dsem):
        c = lax.axis_index("c")
        s = lax.axis_index("s")
        t = c * NS + s

        pltpu.async_copy(ones_hbm, onesv, dsem)
        pltpu.async_copy(zeros_hbm, acc.at[pl.ds(s * SLICE, SLICE)], dsem)
        for g, sz in enumerate(sizes):
            pltpu.async_copy(edge_hbm.at[1, t, pl.ds(offs[g], sz)],
                             dstv[g], dsem)
        pltpu.make_async_copy(ones_hbm, onesv, dsem).wait()
        pltpu.make_async_copy(zeros_hbm,
                              acc.at[pl.ds(s * SLICE, SLICE)], dsem).wait()
        for g, sz in enumerate(sizes):
            pltpu.make_async_copy(edge_hbm.at[1, t, pl.ds(offs[g], sz)],
                                  dstv[g], dsem).wait()
        plsc.subcore_barrier()

        for g, sz in enumerate(sizes):
            pltpu.async_copy(onesv.at[pl.ds(0, sz)], acc.at[dstv[g]], dsem,
                             add=True)
        for g, sz in enumerate(sizes):
            pltpu.make_async_copy(onesv.at[pl.ds(0, sz)], acc.at[dstv[g]],
                                  dsem).wait()
        plsc.subcore_barrier()
        pltpu.sync_copy(acc.at[pl.ds(s * SLICE, SLICE)],
                        out_hbm.at[c, pl.ds(s * SLICE, SLICE)])

    return deg_kernel


def _make_sc_agg(sizes, W):
    offs = [sum(sizes[:g]) for g in range(len(sizes))]
    NSUP = len(sizes)

    @functools.partial(
        pl.kernel,
        out_type=jax.ShapeDtypeStruct((NC, NP, W), jnp.float32),
        mesh=_mesh,
        compiler_params=_sc_params,
        scratch_types=[
            [pltpu.VMEM((sz,), jnp.int32) for sz in sizes],
            [pltpu.VMEM((sz,), jnp.int32) for sz in sizes],
            [pltpu.VMEM((JC, W), jnp.float32) for _ in range(2)],
            pltpu.VMEM_SHARED((NP, W), jnp.float32),
            pltpu.VMEM_SHARED((NP, W), jnp.float32),
            [pltpu.SemaphoreType.DMA for _ in range(2)],
            [pltpu.SemaphoreType.DMA for _ in range(2)],
            pltpu.SemaphoreType.DMA,
        ],
    )
    def agg_kernel(table_hbm, edge_hbm, zeros_hbm, out_hbm,
                   srcv, dstv, rows, acc, tbl, gsem, ssem, isem):
        c = lax.axis_index("c")
        s = lax.axis_index("s")
        t = c * NS + s

        pltpu.sync_copy(zeros_hbm, acc.at[pl.ds(s * SLICE, SLICE)])
        pltpu.async_copy(table_hbm.at[pl.ds(s * SLICE, SLICE)],
                         tbl.at[pl.ds(s * SLICE, SLICE)], isem)
        for g, sz in enumerate(sizes):
            pltpu.async_copy(edge_hbm.at[0, t, pl.ds(offs[g], sz)],
                             srcv[g], isem)
            pltpu.async_copy(edge_hbm.at[1, t, pl.ds(offs[g], sz)],
                             dstv[g], isem)
        for g, sz in enumerate(sizes):
            pltpu.make_async_copy(edge_hbm.at[0, t, pl.ds(offs[g], sz)],
                                  srcv[g], isem).wait()
            pltpu.make_async_copy(edge_hbm.at[1, t, pl.ds(offs[g], sz)],
                                  dstv[g], isem).wait()
        pltpu.make_async_copy(table_hbm.at[pl.ds(s * SLICE, SLICE)],
                              tbl.at[pl.ds(s * SLICE, SLICE)], isem).wait()
        plsc.subcore_barrier()

        def rowbuf(b, sz):
            return rows[b].at[pl.ds(0, sz)] if sz != JC else rows[b]

        pending = [None, None]
        pltpu.async_copy(tbl.at[srcv[0]], rowbuf(0, sizes[0]), gsem[0])
        for g, sz in enumerate(sizes):
            b = g % 2
            o = 1 - b
            pltpu.make_async_copy(tbl.at[srcv[g]], rowbuf(b, sz),
                                  gsem[b]).wait()
            pltpu.async_copy(rowbuf(b, sz), acc.at[dstv[g]], ssem[b],
                             add=True)
            pending[b] = g
            if g + 1 < NSUP:
                if pending[o] is not None:
                    po = pending[o]
                    pltpu.make_async_copy(rowbuf(o, sizes[po]),
                                          acc.at[dstv[po]],
                                          ssem[o]).wait()
                    pending[o] = None
                pltpu.async_copy(tbl.at[srcv[g + 1]],
                                 rowbuf(o, sizes[g + 1]), gsem[o])
        for b in range(2):
            if pending[b] is not None:
                pb = pending[b]
                pltpu.make_async_copy(rowbuf(b, sizes[pb]),
                                      acc.at[dstv[pb]], ssem[b]).wait()
        plsc.subcore_barrier()
        pltpu.sync_copy(acc.at[pl.ds(s * SLICE, SLICE)],
                        out_hbm.at[c, pl.ds(s * SLICE, SLICE)])

    return agg_kernel


def _tc_matmul1(x_pk, W1B):
    NPKX = x_pk.shape[0]

    def body(xr, w1r, hr):
        h = jnp.dot(xr[...], w1r[...], preferred_element_type=jnp.float32)
        hr[0:NPKX, :] = h
        hr[NPKX:NPK, :] = jnp.zeros((NPK - NPKX, 128), jnp.float32)

    return pl.pallas_call(
        body,
        out_shape=jax.ShapeDtypeStruct((NPK, 128), jnp.float32),
    )(x_pk, W1B)


def _tc_scale1(h1_p, deg_p):
    def body(hr, dpr, h1sr, dinvr):
        deg = dpr[0] + dpr[1] + 1.0
        dinv = lax.rsqrt(deg)
        h1sr[...] = hr[...] * dinv
        dinvr[...] = dinv

    return pl.pallas_call(
        body,
        out_shape=[
            jax.ShapeDtypeStruct((NPK, 128), jnp.float32),
            jax.ShapeDtypeStruct((NPK, 128), jnp.float32),
        ],
    )(h1_p, deg_p)


def _tc_mid(agg1, h1s, dinv, W2B8, dinv8, b1P):
    def body(ar, h1r, dr, w2r, d8r, b1ref, outr):
        agg = ar[0] + ar[1]
        o1 = jnp.maximum(dr[...] * (agg + h1r[...]) + b1ref[...], 0.0)
        o1p = o1.reshape(NPK8, 256)
        h2 = jnp.dot(o1p, w2r[...], preferred_element_type=jnp.float32)
        h2s8 = h2 * d8r[...]
        node = (16 * lax.broadcasted_iota(jnp.int32, (NPK8, 128), 0)
                + lax.broadcasted_iota(jnp.int32, (NPK8, 128), 1) // 8)
        outr[...] = jnp.where(node < N, h2s8, 0.0)

    return pl.pallas_call(
        body,
        out_shape=jax.ShapeDtypeStruct((NPK8, 128), jnp.float32),
    )(agg1, h1s, dinv, W2B8, dinv8, b1P)


def _tc_final(agg2, h2s8, dinv8, b2P8):
    def body(ar, h2r, d8r, b2ref, outr):
        outr[...] = d8r[...] * (ar[0] + ar[1] + h2r[...]) + b2ref[...]

    return pl.pallas_call(
        body,
        out_shape=jax.ShapeDtypeStruct((NPK8, 128), jnp.float32),
    )(agg2, h2s8, dinv8, b2P8)


def kernel(x, edge_index, W1, b1, W2, b2):
    E = edge_index.shape[1]
    ei = edge_index.astype(jnp.int32)
    if E % (NW * 8) != 0:
        pad = NW * 8 - E % (NW * 8)
        ei = jnp.pad(ei, ((0, 0), (0, pad)), constant_values=N)
        E += pad
    EPT = E // NW
    sizes = [JC] * (EPT // JC) + ([EPT % JC] if EPT % JC else [])
    edges = ei.reshape(2, NW, EPT)
    x_pk = x.reshape(x.shape[0] // 8, 8 * 128)
    W1B = jnp.kron(jnp.eye(8, dtype=jnp.float32), W1)
    W2p8 = jnp.pad(W2, ((0, 0), (0, 8 - W2.shape[1])))
    W2B8 = jnp.kron(jnp.eye(16, dtype=jnp.float32), W2p8)
    b1P = jnp.tile(b1, 8).reshape(1, 128)
    b2P8 = jnp.tile(jnp.pad(b2, (0, 8 - b2.shape[0])), 16).reshape(1, 128)
    zeros16 = jnp.zeros((SLICE, F), jnp.float32)
    zeros8 = jnp.zeros((SLICE, 8), jnp.float32)
    ones16 = jnp.ones((JC, F), jnp.float32)

    sc_degree = _make_sc_degree(sizes)
    sc_agg16 = _make_sc_agg(sizes, F)
    sc_agg8 = _make_sc_agg(sizes, 8)

    h1_p = _tc_matmul1(x_pk, W1B)
    deg_p = sc_degree(edges, ones16, zeros16).reshape(NC, NPK, 128)
    h1s_p, dinv_p = _tc_scale1(h1_p, deg_p)
    dinv8 = dinv_p.reshape(NP, F)[:, :8].reshape(NPK8, 128)
    agg1 = sc_agg16(h1s_p.reshape(NP, F), edges,
                    zeros16).reshape(NC, NPK, 128)
    h2s8 = _tc_mid(agg1, h1s_p, dinv_p, W2B8, dinv8, b1P)
    agg2 = sc_agg8(h2s8.reshape(NP, 8), edges,
                   zeros8).reshape(NC, NPK8, 128)
    out8 = _tc_final(agg2, h2s8, dinv8, b2P8)
    return out8.reshape(NP, 8)[:N, :W2.shape[1]]


# speedup vs baseline: 1.0228x; 1.0228x over previous
import functools

import jax
import jax.numpy as jnp
from jax import lax
from jax.experimental import pallas as pl
from jax.experimental.pallas import tpu as pltpu
from jax.experimental.pallas import tpu_sc as plsc

N = 10000
NP = 10240
F = 16
NC = 2
NS = 16
NW = NC * NS
JC = 1024
SLICE = NP // NS
NPK = NP // 8
NPK8 = NP // 16

_mesh = plsc.VectorSubcoreMesh(
    core_axis_name="c", subcore_axis_name="s", num_cores=NC, num_subcores=NS)
_sc_params = pltpu.CompilerParams(use_tc_tiling_on_sc=False)


def _fill(ref, n_rows, value):
    def body(j, _):
        ref[j, :] = jnp.full((16,), value, jnp.float32)
        return 0
    lax.fori_loop(0, n_rows, body, 0)


def _make_sc_degree(sizes):
    offs = [sum(sizes[:g]) for g in range(len(sizes))]

    @functools.partial(
        pl.kernel,
        out_type=jax.ShapeDtypeStruct((NC, NP, F), jnp.float32),
        mesh=_mesh,
        compiler_params=_sc_params,
        scratch_types=[
            [pltpu.VMEM((sz,), jnp.int32) for sz in sizes],
            pltpu.VMEM((JC, F), jnp.float32),
            pltpu.VMEM((SLICE, F), jnp.float32),
            pltpu.VMEM_SHARED((NP, F), jnp.float32),
            pltpu.SemaphoreType.DMA,
        ],
    )
    def deg_kernel(edge_hbm, out_hbm, dstv, onesv, zerov, acc, dsem):
        c = lax.axis_index("c")
        s = lax.axis_index("s")
        t = c * NS + s

        _fill(onesv, JC, 1.0)
        _fill(zerov, SLICE, 0.0)
        pltpu.sync_copy(zerov, acc.at[pl.ds(s * SLICE, SLICE)])
        for g, sz in enumerate(sizes):
            pltpu.async_copy(edge_hbm.at[1, t, pl.ds(offs[g], sz)],
                             dstv[g], dsem)
        for g, sz in enumerate(sizes):
            pltpu.make_async_copy(edge_hbm.at[1, t, pl.ds(offs[g], sz)],
                                  dstv[g], dsem).wait()
        plsc.subcore_barrier()

        for g, sz in enumerate(sizes):
            pltpu.async_copy(onesv.at[pl.ds(0, sz)], acc.at[dstv[g]], dsem,
                             add=True)
        for g, sz in enumerate(sizes):
            pltpu.make_async_copy(onesv.at[pl.ds(0, sz)], acc.at[dstv[g]],
                                  dsem).wait()
        plsc.subcore_barrier()
        pltpu.sync_copy(acc.at[pl.ds(s * SLICE, SLICE)],
                        out_hbm.at[c, pl.ds(s * SLICE, SLICE)])

    return deg_kernel


def _make_sc_agg(sizes, W):
    offs = [sum(sizes[:g]) for g in range(len(sizes))]
    NSUP = len(sizes)

    @functools.partial(
        pl.kernel,
        out_type=jax.ShapeDtypeStruct((NC, NP, W), jnp.float32),
        mesh=_mesh,
        compiler_params=_sc_params,
        scratch_types=[
            [pltpu.VMEM((sz,), jnp.int32) for sz in sizes],
            [pltpu.VMEM((sz,), jnp.int32) for sz in sizes],
            [pltpu.VMEM((JC, W), jnp.float32) for _ in range(2)],
            pltpu.VMEM_SHARED((NP, W), jnp.float32),
            pltpu.VMEM_SHARED((NP, W), jnp.float32),
            [pltpu.SemaphoreType.DMA for _ in range(2)],
            [pltpu.SemaphoreType.DMA for _ in range(2)],
            pltpu.SemaphoreType.DMA,
        ],
    )
    def agg_kernel(table_hbm, edge_hbm, zeros_hbm, out_hbm,
                   srcv, dstv, rows, acc, tbl, gsem, ssem, isem):
        c = lax.axis_index("c")
        s = lax.axis_index("s")
        t = c * NS + s

        pltpu.sync_copy(zeros_hbm, acc.at[pl.ds(s * SLICE, SLICE)])
        pltpu.async_copy(table_hbm.at[pl.ds(s * SLICE, SLICE)],
                         tbl.at[pl.ds(s * SLICE, SLICE)], isem)
        for g, sz in enumerate(sizes):
            pltpu.async_copy(edge_hbm.at[0, t, pl.ds(offs[g], sz)],
                             srcv[g], isem)
            pltpu.async_copy(edge_hbm.at[1, t, pl.ds(offs[g], sz)],
                             dstv[g], isem)
        for g, sz in enumerate(sizes):
            pltpu.make_async_copy(edge_hbm.at[0, t, pl.ds(offs[g], sz)],
                                  srcv[g], isem).wait()
            pltpu.make_async_copy(edge_hbm.at[1, t, pl.ds(offs[g], sz)],
                                  dstv[g], isem).wait()
        pltpu.make_async_copy(table_hbm.at[pl.ds(s * SLICE, SLICE)],
                              tbl.at[pl.ds(s * SLICE, SLICE)], isem).wait()
        plsc.subcore_barrier()

        def rowbuf(b, sz):
            return rows[b].at[pl.ds(0, sz)] if sz != JC else rows[b]

        pending = [None, None]
        pltpu.async_copy(tbl.at[srcv[0]], rowbuf(0, sizes[0]), gsem[0])
        for g, sz in enumerate(sizes):
            b = g % 2
            o = 1 - b
            pltpu.make_async_copy(tbl.at[srcv[g]], rowbuf(b, sz),
                                  gsem[b]).wait()
            pltpu.async_copy(rowbuf(b, sz), acc.at[dstv[g]], ssem[b],
                             add=True)
            pending[b] = g
            if g + 1 < NSUP:
                if pending[o] is not None:
                    po = pending[o]
                    pltpu.make_async_copy(rowbuf(o, sizes[po]),
                                          acc.at[dstv[po]],
                                          ssem[o]).wait()
                    pending[o] = None
                pltpu.async_copy(tbl.at[srcv[g + 1]],
                                 rowbuf(o, sizes[g + 1]), gsem[o])
        for b in range(2):
            if pending[b] is not None:
                pb = pending[b]
                pltpu.make_async_copy(rowbuf(b, sizes[pb]),
                                      acc.at[dstv[pb]], ssem[b]).wait()
        plsc.subcore_barrier()
        pltpu.sync_copy(acc.at[pl.ds(s * SLICE, SLICE)],
                        out_hbm.at[c, pl.ds(s * SLICE, SLICE)])

    return agg_kernel


def _tc_matmul1(x_pk, W1B):
    NPKX = x_pk.shape[0]

    def body(xr, w1r, hr):
        h = jnp.dot(xr[...], w1r[...], preferred_element_type=jnp.float32)
        hr[0:NPKX, :] = h
        hr[NPKX:NPK, :] = jnp.zeros((NPK - NPKX, 128), jnp.float32)

    return pl.pallas_call(
        body,
        out_shape=jax.ShapeDtypeStruct((NPK, 128), jnp.float32),
    )(x_pk, W1B)


def _tc_scale1(h1_p, deg_p):
    def body(hr, dpr, h1sr, dinvr):
        deg = dpr[0] + dpr[1] + 1.0
        dinv = lax.rsqrt(deg)
        h1sr[...] = hr[...] * dinv
        dinvr[...] = dinv

    return pl.pallas_call(
        body,
        out_shape=[
            jax.ShapeDtypeStruct((NPK, 128), jnp.float32),
            jax.ShapeDtypeStruct((NPK, 128), jnp.float32),
        ],
    )(h1_p, deg_p)


def _tc_mid(agg1, h1s, dinv, W2B8, dinv8, b1P):
    def body(ar, h1r, dr, w2r, d8r, b1ref, outr):
        agg = ar[0] + ar[1]
        o1 = jnp.maximum(dr[...] * (agg + h1r[...]) + b1ref[...], 0.0)
        o1p = o1.reshape(NPK8, 256)
        h2 = jnp.dot(o1p, w2r[...], preferred_element_type=jnp.float32)
        h2s8 = h2 * d8r[...]
        node = (16 * lax.broadcasted_iota(jnp.int32, (NPK8, 128), 0)
                + lax.broadcasted_iota(jnp.int32, (NPK8, 128), 1) // 8)
        outr[...] = jnp.where(node < N, h2s8, 0.0)

    return pl.pallas_call(
        body,
        out_shape=jax.ShapeDtypeStruct((NPK8, 128), jnp.float32),
    )(agg1, h1s, dinv, W2B8, dinv8, b1P)


def _tc_final(agg2, h2s8, dinv8, b2P8):
    def body(ar, h2r, d8r, b2ref, outr):
        outr[...] = d8r[...] * (ar[0] + ar[1] + h2r[...]) + b2ref[...]

    return pl.pallas_call(
        body,
        out_shape=jax.ShapeDtypeStruct((NPK8, 128), jnp.float32),
    )(agg2, h2s8, dinv8, b2P8)


def kernel(x, edge_index, W1, b1, W2, b2):
    E = edge_index.shape[1]
    ei = edge_index.astype(jnp.int32)
    if E % (NW * 8) != 0:
        pad = NW * 8 - E % (NW * 8)
        ei = jnp.pad(ei, ((0, 0), (0, pad)), constant_values=N)
        E += pad
    EPT = E // NW
    sizes = [JC] * (EPT // JC) + ([EPT % JC] if EPT % JC else [])
    edges = ei.reshape(2, NW, EPT)
    x_pk = x.reshape(x.shape[0] // 8, 8 * 128)
    W1B = jnp.kron(jnp.eye(8, dtype=jnp.float32), W1)
    W2p8 = jnp.pad(W2, ((0, 0), (0, 8 - W2.shape[1])))
    W2B8 = jnp.kron(jnp.eye(16, dtype=jnp.float32), W2p8)
    b1P = jnp.tile(b1, 8).reshape(1, 128)
    b2P8 = jnp.tile(jnp.pad(b2, (0, 8 - b2.shape[0])), 16).reshape(1, 128)
    zeros16 = jnp.zeros((SLICE, F), jnp.float32)
    zeros8 = jnp.zeros((SLICE, 8), jnp.float32)

    sc_degree = _make_sc_degree(sizes)
    sc_agg16 = _make_sc_agg(sizes, F)
    sc_agg8 = _make_sc_agg(sizes, 8)

    h1_p = _tc_matmul1(x_pk, W1B)
    deg_p = sc_degree(edges).reshape(NC, NPK, 128)
    h1s_p, dinv_p = _tc_scale1(h1_p, deg_p)
    dinv8 = dinv_p.reshape(NP, F)[:, :8].reshape(NPK8, 128)
    agg1 = sc_agg16(h1s_p.reshape(NP, F), edges,
                    zeros16).reshape(NC, NPK, 128)
    h2s8 = _tc_mid(agg1, h1s_p, dinv_p, W2B8, dinv8, b1P)
    agg2 = sc_agg8(h2s8.reshape(NP, 8), edges,
                   zeros8).reshape(NC, NPK8, 128)
    out8 = _tc_final(agg2, h2s8, dinv8, b2P8)
    return out8.reshape(NP, 8)[:N, :W2.shape[1]]
